# baseline (speedup 1.0000x reference)
import functools

import jax
import jax.numpy as jnp
from jax import lax
from jax.experimental import pallas as pl
from jax.experimental.pallas import tpu as pltpu
from jax.experimental.pallas import tpu_sc as plsc

_NUM_CP = 65536
_DX = 2.0 / (_NUM_CP - 3)

_NC = 2
_NS = 16
_NW = _NC * _NS
_LANES = 16

_TAB_LO = 32760
_TAB_N = _NUM_CP - _TAB_LO
_TAB_ALLOC = _TAB_N + _LANES

_CHUNK = 16384
_VECS = _CHUNK // _LANES


def _spline_body(t_hbm, phi_hbm, out_hbm, phi_v, w0_tab, w1_tab, t_bufs,
                 o_bufs, sems, n_per_w):
    wid = lax.axis_index("s") * _NC + lax.axis_index("c")
    base = wid * n_per_w

    rdxf = jnp.float32(1.0) / jnp.float32(_DX)
    n_chunks = n_per_w // _CHUNK
    n_super = n_chunks // 2
    in_sems, out_sems = sems

    def in_slice(c):
        return t_hbm.at[pl.ds(base + c * _CHUNK, _CHUNK)]

    def out_slice(c):
        return out_hbm.at[pl.ds(base + c * _CHUNK, _CHUNK)]

    for b in range(2):
        pltpu.async_copy(in_slice(b), t_bufs[b], in_sems[b])
    pltpu.sync_copy(phi_hbm.at[pl.ds(_TAB_LO, _TAB_N)],
                    phi_v.at[pl.ds(0, _TAB_N)])
    phi_v[pl.ds(_TAB_N, _LANES)] = jnp.zeros((_LANES,), jnp.float32)

    lane_f = lax.iota(jnp.int32, _LANES).astype(jnp.float32)

    def wtab_body(j, carry):
        u = (lane_f + j.astype(jnp.float32) * 16.0) * jnp.float32(1.0 / 512.0)
        w0 = (
            (u * jnp.float32(-1.0 / 6.0) + jnp.float32(0.5)) * u
            - jnp.float32(0.5)
        ) * u + jnp.float32(1.0 / 6.0)
        w1 = (u * jnp.float32(0.5) - jnp.float32(1.0)) * (u * u) + jnp.float32(
            2.0 / 3.0
        )
        w0_tab[pl.ds(j * _LANES, _LANES)] = w0
        w1_tab[pl.ds(j * _LANES, _LANES)] = w1
        return carry

    lax.fori_loop(0, 512 // _LANES, wtab_body, 0)

    def compute(t_v, o_v):
        @plsc.parallel_loop(0, _VECS, 1, unroll=8)
        def _vec(vi):
            tv = t_v[pl.ds(vi * _LANES, _LANES)]
            q = tv * rdxf + rdxf
            idx = q.astype(jnp.int32)
            u = q - idx.astype(jnp.float32)
            iloc = idx - _TAB_LO
            ui = (u * jnp.float32(512.0)).astype(jnp.int32)
            g0 = plsc.load_gather(phi_v, [iloc])
            g1 = plsc.load_gather(phi_v, [iloc + 1])
            g2 = plsc.load_gather(phi_v, [iloc + 2])
            g3 = plsc.load_gather(phi_v, [iloc + 3])
            w0 = plsc.load_gather(w0_tab, [ui])
            w1 = plsc.load_gather(w1_tab, [ui])
            w3 = (u * jnp.float32(1.0 / 6.0)) * (u * u)
            acc = w0 * (g0 - g2) + g2
            acc = w1 * (g1 - g2) + acc
            acc = w3 * (g3 - g2) + acc
            o_v[pl.ds(vi * _LANES, _LANES)] = acc

    def super_body(si, carry):
        for b in range(2):
            c = si * 2 + b
            pltpu.make_async_copy(in_slice(c), t_bufs[b], in_sems[b]).wait()
            compute(t_bufs[b], o_bufs[b])
            @pl.when(si > 0)
            def _():
                pltpu.make_async_copy(
                    o_bufs[b], out_slice(c), out_sems[b]
                ).wait()
            pltpu.async_copy(o_bufs[b], out_slice(c), out_sems[b])
            c_next = jnp.where(si + 1 < n_super, c + 2, b)
            pltpu.async_copy(in_slice(c_next), t_bufs[b], in_sems[b])
        return carry

    lax.fori_loop(0, n_super, super_body, 0)

    for b in range(2):
        pltpu.make_async_copy(in_slice(b), t_bufs[b], in_sems[b]).wait()
        pltpu.make_async_copy(o_bufs[b], out_slice(b), out_sems[b]).wait()


def kernel(_t, phi_x):
    n = _t.shape[0]
    assert n % (_NW * 2 * _CHUNK) == 0
    n_per_w = n // _NW

    mesh = plsc.VectorSubcoreMesh(core_axis_name="c", subcore_axis_name="s")
    f = pl.kernel(
        functools.partial(_spline_body, n_per_w=n_per_w),
        out_type=jax.ShapeDtypeStruct((n,), jnp.float32),
        mesh=mesh,
        scratch_types=[
            pltpu.VMEM((_TAB_ALLOC,), jnp.float32),
            pltpu.VMEM((512,), jnp.float32),
            pltpu.VMEM((512,), jnp.float32),
            [pltpu.VMEM((_CHUNK,), jnp.float32) for _ in range(2)],
            [pltpu.VMEM((_CHUNK,), jnp.float32) for _ in range(2)],
            ([pltpu.SemaphoreType.DMA for _ in range(2)],
             [pltpu.SemaphoreType.DMA for _ in range(2)]),
        ],
        compiler_params=pltpu.CompilerParams(needs_layout_passes=False),
    )
    return f(_t, phi_x)


# speedup vs baseline: 1.4802x; 1.4802x over previous
import functools

import jax
import jax.numpy as jnp
from jax import lax
from jax.experimental import pallas as pl
from jax.experimental.pallas import tpu as pltpu
from jax.experimental.pallas import tpu_sc as plsc

_NUM_CP = 65536
_DX = 2.0 / (_NUM_CP - 3)

_NC = 2
_NS = 16
_NW = _NC * _NS
_LANES = 16

_TAB_LO = 32760
_TAB_N = _NUM_CP - _TAB_LO
_TAB_ALLOC = _TAB_N + _LANES

_CHUNK = 16384
_VECS = _CHUNK // _LANES


def _spline_body(t_hbm, phi_hbm, out_hbm, phi_v, t_bufs, o_bufs, sems,
                 n_per_w):
    wid = lax.axis_index("s") * _NC + lax.axis_index("c")
    base = wid * n_per_w

    rdxf = jnp.float32(1.0) / jnp.float32(_DX)
    c0 = rdxf - jnp.float32(_TAB_LO)
    n_chunks = n_per_w // _CHUNK
    n_super = n_chunks // 2
    in_sems, out_sems = sems

    def in_slice(c):
        return t_hbm.at[pl.ds(base + c * _CHUNK, _CHUNK)]

    def out_slice(c):
        return out_hbm.at[pl.ds(base + c * _CHUNK, _CHUNK)]

    for b in range(2):
        pltpu.async_copy(in_slice(b), t_bufs[b], in_sems[b])
    pltpu.sync_copy(phi_hbm.at[pl.ds(_TAB_LO, _TAB_N)],
                    phi_v.at[pl.ds(0, _TAB_N)])
    phi_v[pl.ds(_TAB_N, _LANES)] = jnp.zeros((_LANES,), jnp.float32)

    def compute(t_v, o_v):
        @plsc.parallel_loop(0, _VECS, 1, unroll=6)
        def _vec(vi):
            tv = t_v[pl.ds(vi * _LANES, _LANES)]
            q = tv * rdxf + c0
            idx = q.astype(jnp.int32)
            u = q - idx.astype(jnp.float32)
            g0 = plsc.load_gather(phi_v, [idx])
            g1 = plsc.load_gather(phi_v, [idx + 1])
            g2 = plsc.load_gather(phi_v, [idx + 2])
            g3 = plsc.load_gather(phi_v, [idx + 3])
            u2 = u * u
            w0 = (
                (u * jnp.float32(-1.0 / 6.0) + jnp.float32(0.5)) * u
                - jnp.float32(0.5)
            ) * u + jnp.float32(1.0 / 6.0)
            w1 = (u * jnp.float32(0.5) - jnp.float32(1.0)) * u2 + jnp.float32(
                2.0 / 3.0
            )
            w3 = (u * jnp.float32(1.0 / 6.0)) * u2
            acc = w0 * (g0 - g2) + g2
            acc = w1 * (g1 - g2) + acc
            acc = w3 * (g3 - g2) + acc
            o_v[pl.ds(vi * _LANES, _LANES)] = acc

    def super_body(si, carry):
        for b in range(2):
            c = si * 2 + b
            pltpu.make_async_copy(in_slice(c), t_bufs[b], in_sems[b]).wait()
            compute(t_bufs[b], o_bufs[b])
            @pl.when(si > 0)
            def _():
                pltpu.make_async_copy(
                    o_bufs[b], out_slice(c), out_sems[b]
                ).wait()
            pltpu.async_copy(o_bufs[b], out_slice(c), out_sems[b])
            c_next = jnp.where(si + 1 < n_super, c + 2, b)
            pltpu.async_copy(in_slice(c_next), t_bufs[b], in_sems[b])
        return carry

    lax.fori_loop(0, n_super, super_body, 0)

    for b in range(2):
        pltpu.make_async_copy(in_slice(b), t_bufs[b], in_sems[b]).wait()
        pltpu.make_async_copy(o_bufs[b], out_slice(b), out_sems[b]).wait()


def kernel(_t, phi_x):
    n = _t.shape[0]
    assert n % (_NW * 2 * _CHUNK) == 0
    n_per_w = n // _NW

    mesh = plsc.VectorSubcoreMesh(core_axis_name="c", subcore_axis_name="s")
    f = pl.kernel(
        functools.partial(_spline_body, n_per_w=n_per_w),
        out_type=jax.ShapeDtypeStruct((n,), jnp.float32),
        mesh=mesh,
        scratch_types=[
            pltpu.VMEM((_TAB_ALLOC,), jnp.float32),
            [pltpu.VMEM((_CHUNK,), jnp.float32) for _ in range(2)],
            [pltpu.VMEM((_CHUNK,), jnp.float32) for _ in range(2)],
            ([pltpu.SemaphoreType.DMA for _ in range(2)],
             [pltpu.SemaphoreType.DMA for _ in range(2)]),
        ],
        compiler_params=pltpu.CompilerParams(needs_layout_passes=False),
    )
    return f(_t, phi_x)


# speedup vs baseline: 1.4906x; 1.0070x over previous
import functools

import jax
import jax.numpy as jnp
from jax import lax
from jax.experimental import pallas as pl
from jax.experimental.pallas import tpu as pltpu
from jax.experimental.pallas import tpu_sc as plsc

_NUM_CP = 65536
_DX = 2.0 / (_NUM_CP - 3)

_NC = 2
_NS = 16
_NW = _NC * _NS
_LANES = 16

_TAB_LO = 32760
_TAB_N = _NUM_CP - _TAB_LO
_TAB_ALLOC = _TAB_N + _LANES

_CHUNK = 16384
_VECS = _CHUNK // _LANES


def _spline_body(t_hbm, phi_hbm, out_hbm, phi_v, t_bufs, o_bufs, sems,
                 n_per_w):
    wid = lax.axis_index("s") * _NC + lax.axis_index("c")
    base = wid * n_per_w

    rdxf = jnp.float32(1.0) / jnp.float32(_DX)
    c0 = rdxf - jnp.float32(_TAB_LO)
    n_chunks = n_per_w // _CHUNK
    n_super = n_chunks // 2
    in_sems, out_sems = sems

    def in_slice(c):
        return t_hbm.at[pl.ds(base + c * _CHUNK, _CHUNK)]

    def out_slice(c):
        return out_hbm.at[pl.ds(base + c * _CHUNK, _CHUNK)]

    for b in range(2):
        pltpu.async_copy(in_slice(b), t_bufs[b], in_sems[b])
    pltpu.sync_copy(phi_hbm.at[pl.ds(_TAB_LO, _TAB_N)],
                    phi_v.at[pl.ds(0, _TAB_N)])
    phi_v[pl.ds(_TAB_N, _LANES)] = jnp.zeros((_LANES,), jnp.float32)

    def compute(t_v, o_v):
        @plsc.parallel_loop(0, _VECS, 1, unroll=8)
        def _vec(vi):
            tv = t_v[pl.ds(vi * _LANES, _LANES)]
            q = tv * rdxf + c0
            idx = q.astype(jnp.int32)
            u = q - idx.astype(jnp.float32)
            g0 = plsc.load_gather(phi_v, [idx])
            g1 = plsc.load_gather(phi_v, [idx + 1])
            g2 = plsc.load_gather(phi_v, [idx + 2])
            g3 = plsc.load_gather(phi_v, [idx + 3])
            u2 = u * u
            w0 = (
                (u * jnp.float32(-1.0 / 6.0) + jnp.float32(0.5)) * u
                - jnp.float32(0.5)
            ) * u + jnp.float32(1.0 / 6.0)
            w1 = (u * jnp.float32(0.5) - jnp.float32(1.0)) * u2 + jnp.float32(
                2.0 / 3.0
            )
            w3 = (u * jnp.float32(1.0 / 6.0)) * u2
            acc = w0 * (g0 - g2) + g2
            acc = w1 * (g1 - g2) + acc
            acc = w3 * (g3 - g2) + acc
            o_v[pl.ds(vi * _LANES, _LANES)] = acc

    def super_body(si, carry):
        for b in range(2):
            c = si * 2 + b
            pltpu.make_async_copy(in_slice(c), t_bufs[b], in_sems[b]).wait()
            compute(t_bufs[b], o_bufs[b])
            @pl.when(si > 0)
            def _():
                pltpu.make_async_copy(
                    o_bufs[b], out_slice(c), out_sems[b]
                ).wait()
            pltpu.async_copy(o_bufs[b], out_slice(c), out_sems[b])
            c_next = jnp.where(si + 1 < n_super, c + 2, b)
            pltpu.async_copy(in_slice(c_next), t_bufs[b], in_sems[b])
        return carry

    lax.fori_loop(0, n_super, super_body, 0)

    for b in range(2):
        pltpu.make_async_copy(in_slice(b), t_bufs[b], in_sems[b]).wait()
        pltpu.make_async_copy(o_bufs[b], out_slice(b), out_sems[b]).wait()


def kernel(_t, phi_x):
    n = _t.shape[0]
    assert n % (_NW * 2 * _CHUNK) == 0
    n_per_w = n // _NW

    mesh = plsc.VectorSubcoreMesh(core_axis_name="c", subcore_axis_name="s")
    f = pl.kernel(
        functools.partial(_spline_body, n_per_w=n_per_w),
        out_type=jax.ShapeDtypeStruct((n,), jnp.float32),
        mesh=mesh,
        scratch_types=[
            pltpu.VMEM((_TAB_ALLOC,), jnp.float32),
            [pltpu.VMEM((_CHUNK,), jnp.float32) for _ in range(2)],
            [pltpu.VMEM((_CHUNK,), jnp.float32) for _ in range(2)],
            ([pltpu.SemaphoreType.DMA for _ in range(2)],
             [pltpu.SemaphoreType.DMA for _ in range(2)]),
        ],
        compiler_params=pltpu.CompilerParams(needs_layout_passes=False),
    )
    return f(_t, phi_x)
